# Initial kernel scaffold; baseline (speedup 1.0000x reference)
import jax
import jax.numpy as jnp
from jax.experimental import pallas as pl


def kernel(x, edge_index, batch, W1_0, b1_0, W2_0, b2_0, W1_1, b1_1, W2_1, b2_1, W1_2, b1_2, W2_2, b2_2, W1_3, b1_3, W2_3, b2_3, Wl, bl):
    raise NotImplementedError("write your pallas kernel here")


# speedup vs baseline: 4.0358x; 4.0358x over previous
import functools

import jax
import jax.numpy as jnp
from jax import lax
from jax.experimental import pallas as pl
from jax.experimental.pallas import tpu as pltpu
from jax.experimental.pallas import tpu_sc as plsc

N = 100000
H = 128
G = 64
CW = 16
NCH = H // CW
E = 1600000
EPAD = 1605632
BATCHES = EPAD // 128
BPS = BATCHES // 32
GRP = 8
GROUPS = BPS // GRP
ACC_ROWS = 100096
ZROWS = 782
BN = 1000
NB = N // BN
NEG = 0.01


def _leaky(t):
    return jnp.where(t >= 0, t, NEG * t)


def _hi(a, b):
    return lax.dot_general(a, b, (((1,), (0,)), ((), ())),
                           precision=lax.Precision.HIGHEST)


def _make_agg():
    mesh = plsc.VectorSubcoreMesh(core_axis_name="c", subcore_axis_name="s")

    @functools.partial(
        pl.kernel,
        mesh=mesh,
        out_type=jax.ShapeDtypeStruct((2 * N, CW), jnp.float32),
        scratch_types=[
            pltpu.VMEM((GRP, 128), jnp.int32),
            pltpu.VMEM((GRP, 128), jnp.int32),
            pltpu.VMEM((GRP * 128, CW), jnp.float32),
            pltpu.VMEM((ZROWS, CW), jnp.float32),
            pltpu.VMEM_SHARED((ACC_ROWS, CW), jnp.float32),
            pltpu.SemaphoreType.DMA,
        ],
        compiler_params=pltpu.CompilerParams(use_tc_tiling_on_sc=False),
    )
    def agg(table, srcb, dstb, out, src_v, dst_v, rows_v, zero_v, acc, sem):
        c = lax.axis_index("c")
        s = lax.axis_index("s")
        wid = s * 2 + c

        def zloop(i, carry):
            zero_v[i] = jnp.zeros((CW,), jnp.float32)
            return carry

        lax.fori_loop(0, ZROWS, zloop, 0)
        for p in range(8):
            pltpu.sync_copy(zero_v, acc.at[pl.ds(s * 6256 + p * ZROWS, ZROWS)])
        plsc.subcore_barrier()

        base = wid * BPS

        def body(g, carry):
            gb = base + g * GRP
            pltpu.sync_copy(srcb.at[pl.ds(gb, GRP)], src_v)
            pltpu.sync_copy(dstb.at[pl.ds(gb, GRP)], dst_v)
            cps = [
                pltpu.async_copy(
                    table.at[src_v.at[b]],
                    rows_v.at[pl.ds(b * 128, 128)], sem)
                for b in range(GRP)
            ]
            for cp in cps:
                cp.wait()
            for b in range(GRP):
                pltpu.sync_copy(rows_v.at[pl.ds(b * 128, 128)],
                                acc.at[dst_v.at[b]], add=True)
            return carry

        lax.fori_loop(0, GROUPS, body, 0)
        plsc.subcore_barrier()
        @pl.when(s < 15)
        def _wb_main():
            pltpu.sync_copy(acc.at[pl.ds(s * 6256, 6256)],
                            out.at[pl.ds(c * N + s * 6256, 6256)])

        @pl.when(s == 15)
        def _wb_tail():
            pltpu.sync_copy(acc.at[pl.ds(15 * 6256, 6160)],
                            out.at[pl.ds(c * N + 15 * 6256, 6160)])

    return agg


_AGG_CACHE = []


def _agg_call(table, srcb, dstb):
    if not _AGG_CACHE:
        _AGG_CACHE.append(_make_agg())
    return _AGG_CACHE[0](table, srcb, dstb)


def _plo_spec():
    return pl.BlockSpec((BN, CW), lambda i: (i, 0))


def _phi_spec():
    return pl.BlockSpec((BN, CW), lambda i: (i + NB, 0))


def _row_spec():
    return pl.BlockSpec((BN, CW), lambda i: (i, 0))


def _full_spec(shape):
    return pl.BlockSpec(shape, lambda i: tuple(0 for _ in shape))


def _mlp0_body(plo_ref, phi_ref, x_ref, w1_ref, b1_ref, w2_ref, b2_ref,
               *o_refs):
    cat = plo_ref[...] + phi_ref[...] + x_ref[...]
    t = _hi(cat, w1_ref[...]) + b1_ref[...]
    t = _leaky(t)
    t = _hi(t, w2_ref[...]) + b2_ref[...]
    t = _leaky(t)
    for cc in range(NCH):
        o_refs[cc][...] = t[:, cc * CW:(cc + 1) * CW]


def _mlp_body(trailing, *refs):
    ps = refs[0:2 * NCH]
    hs = refs[2 * NCH:3 * NCH]
    w1_ref, b1_ref, w2_ref, b2_ref = refs[3 * NCH:3 * NCH + 4]
    o_refs = refs[3 * NCH + 4:]
    cat = jnp.concatenate(
        [ps[2 * cc][...] + ps[2 * cc + 1][...] + hs[cc][...]
         for cc in range(NCH)], axis=1)
    t = _hi(cat, w1_ref[...]) + b1_ref[...]
    t = _leaky(t)
    t = _hi(t, w2_ref[...]) + b2_ref[...]
    if trailing:
        t = _leaky(t)
    for cc in range(NCH):
        o_refs[cc][...] = t[:, cc * CW:(cc + 1) * CW]


def _final_body(*refs):
    ps = refs[0:2 * NCH]
    hs = refs[2 * NCH:3 * NCH]
    w1_ref, b1_ref, w2_ref, b2_ref, batch_ref, wl_ref, bl_ref = \
        refs[3 * NCH:3 * NCH + 7]
    o_ref = refs[3 * NCH + 7]
    sums, counts = refs[3 * NCH + 8:]
    i = pl.program_id(0)

    @pl.when(i == 0)
    def _init():
        sums[...] = jnp.zeros_like(sums)
        counts[...] = jnp.zeros_like(counts)

    cat = jnp.concatenate(
        [ps[2 * cc][...] + ps[2 * cc + 1][...] + hs[cc][...]
         for cc in range(NCH)], axis=1)
    t = _hi(cat, w1_ref[...]) + b1_ref[...]
    t = _leaky(t)
    t = _hi(t, w2_ref[...]) + b2_ref[...]
    z = jnp.maximum(t, 0.0)
    b = batch_ref[0, 0, :]
    onehot = (b[:, None] ==
              lax.broadcasted_iota(jnp.int32, (BN, G), 1)).astype(jnp.float32)
    sums[...] += lax.dot_general(onehot, z, (((0,), (0,)), ((), ())),
                                 precision=lax.Precision.HIGHEST)
    counts[...] += lax.dot_general(onehot, jnp.ones((BN, 1), jnp.float32),
                                   (((0,), (0,)), ((), ())),
                                   precision=lax.Precision.HIGHEST)

    @pl.when(i == NB - 1)
    def _fin():
        mean = sums[...] / jnp.maximum(counts[...], 1.0)
        logits = _hi(mean, wl_ref[...]) + bl_ref[...]
        m = jnp.max(logits, axis=1, keepdims=True)
        lse = m + jnp.log(jnp.sum(jnp.exp(logits - m), axis=1, keepdims=True))
        o_ref[...] = logits - lse


def _mlp0_call(p0, x16, w1e, b1, w2, b2):
    return pl.pallas_call(
        _mlp0_body,
        grid=(NB,),
        in_specs=[
            _plo_spec(), _phi_spec(), _row_spec(),
            _full_spec((CW, H)), _full_spec((1, H)),
            _full_spec((H, H)), _full_spec((1, H)),
        ],
        out_specs=[_row_spec() for _ in range(NCH)],
        out_shape=[jax.ShapeDtypeStruct((N, CW), jnp.float32)
                   for _ in range(NCH)],
    )(p0, p0, x16, w1e, b1, w2, b2)


def _mlp_call(ps, hs, w1, b1, w2, b2, trailing):
    return pl.pallas_call(
        functools.partial(_mlp_body, trailing),
        grid=(NB,),
        in_specs=(
            [sp for _ in range(NCH) for sp in (_plo_spec(), _phi_spec())]
            + [_row_spec() for _ in range(NCH)]
            + [_full_spec((H, H)), _full_spec((1, H)),
               _full_spec((H, H)), _full_spec((1, H))]
        ),
        out_specs=[_row_spec() for _ in range(NCH)],
        out_shape=[jax.ShapeDtypeStruct((N, CW), jnp.float32)
                   for _ in range(NCH)],
    )(*[p for pp in ps for p in (pp, pp)], *hs, w1, b1, w2, b2)


def _final_call(ps, hs, w1, b1, w2, b2, batch3d, wl, bl):
    return pl.pallas_call(
        _final_body,
        grid=(NB,),
        in_specs=(
            [sp for _ in range(NCH) for sp in (_plo_spec(), _phi_spec())]
            + [_row_spec() for _ in range(NCH)]
            + [_full_spec((H, H)), _full_spec((1, H)),
               _full_spec((H, H)), _full_spec((1, H)),
               pl.BlockSpec((1, 1, BN), lambda i: (i, 0, 0)),
               _full_spec((H, 2)), _full_spec((1, 2))]
        ),
        out_specs=pl.BlockSpec((G, 2), lambda i: (0, 0)),
        out_shape=jax.ShapeDtypeStruct((G, 2), jnp.float32),
        scratch_shapes=[pltpu.VMEM((G, H), jnp.float32),
                        pltpu.VMEM((G, 1), jnp.float32)],
        compiler_params=pltpu.CompilerParams(
            dimension_semantics=("arbitrary",)),
    )(*[p for pp in ps for p in (pp, pp)], *hs, w1, b1, w2, b2,
      batch3d, wl, bl)


def kernel(x, edge_index, batch, W1_0, b1_0, W2_0, b2_0, W1_1, b1_1, W2_1,
           b2_1, W1_2, b1_2, W2_2, b2_2, W1_3, b1_3, W2_3, b2_3, Wl, bl):
    src = edge_index[0]
    dst = edge_index[1]
    pad = EPAD - E
    srcb = jnp.concatenate(
        [src, jnp.zeros((pad,), jnp.int32)]).reshape(BATCHES, 128)
    dstb = jnp.concatenate(
        [dst, jnp.full((pad,), N, jnp.int32)]).reshape(BATCHES, 128)
    x16 = jnp.tile(x, (1, CW))
    batch3d = batch.reshape(NB, 1, BN)
    w1e = jnp.concatenate([W1_0, jnp.zeros((CW - 1, H), jnp.float32)], axis=0)
    rb = lambda v: v.reshape(1, -1)

    p0 = _agg_call(x16, srcb, dstb)
    hs = _mlp0_call(p0, x16, w1e, rb(b1_0), W2_0, rb(b2_0))
    for (w1, b1, w2, b2) in ((W1_1, b1_1, W2_1, b2_1),
                             (W1_2, b1_2, W2_2, b2_2)):
        ps = [_agg_call(hs[cc], srcb, dstb) for cc in range(NCH)]
        hs = _mlp_call(ps, hs, w1, rb(b1), w2, rb(b2), trailing=True)
    ps = [_agg_call(hs[cc], srcb, dstb) for cc in range(NCH)]
    return _final_call(ps, hs, W1_3, rb(b1_3), W2_3, rb(b2_3),
                       batch3d, Wl, rb(bl))


# speedup vs baseline: 4.9659x; 1.2305x over previous
import functools

import jax
import jax.numpy as jnp
from jax import lax
from jax.experimental import pallas as pl
from jax.experimental.pallas import tpu as pltpu
from jax.experimental.pallas import tpu_sc as plsc

N = 100000
H = 128
G = 64
CW = 16
NCH = H // CW
E = 1600000
EPAD = 1605632
BATCHES = EPAD // 128
BPS = BATCHES // 32
GRP = 4
GROUPS = BPS // GRP
ACC_ROWS = 100096
ZROWS = 782
BN = 1000
NB = N // BN
NEG = 0.01


def _leaky(t):
    return jnp.where(t >= 0, t, NEG * t)


def _hi(a, b):
    return lax.dot_general(a, b, (((1,), (0,)), ((), ())),
                           precision=lax.Precision.HIGHEST)


def _make_agg():
    mesh = plsc.VectorSubcoreMesh(core_axis_name="c", subcore_axis_name="s")

    @functools.partial(
        pl.kernel,
        mesh=mesh,
        out_type=jax.ShapeDtypeStruct((2 * N, CW), jnp.float32),
        scratch_types=[
            pltpu.VMEM((2, GRP, 128), jnp.int32),
            pltpu.VMEM((2, GRP, 128), jnp.int32),
            pltpu.VMEM((2, GRP * 128, CW), jnp.float32),
            pltpu.VMEM((ZROWS, CW), jnp.float32),
            pltpu.VMEM_SHARED((ACC_ROWS, CW), jnp.float32),
            pltpu.SemaphoreType.DMA,
            pltpu.SemaphoreType.DMA,
            pltpu.SemaphoreType.DMA,
            pltpu.SemaphoreType.DMA,
            pltpu.SemaphoreType.DMA,
            pltpu.SemaphoreType.DMA,
        ],
        compiler_params=pltpu.CompilerParams(use_tc_tiling_on_sc=False),
    )
    def agg(table, srcb, dstb, out, src_v, dst_v, rows_v, zero_v, acc,
            isem0, isem1, gsem0, gsem1, ssem0, ssem1):
        c = lax.axis_index("c")
        s = lax.axis_index("s")
        wid = s * 2 + c
        isem = (isem0, isem1)
        gsem = (gsem0, gsem1)
        ssem = (ssem0, ssem1)

        def zloop(i, carry):
            zero_v[i] = jnp.zeros((CW,), jnp.float32)
            return carry

        lax.fori_loop(0, ZROWS, zloop, 0)
        for p in range(8):
            pltpu.sync_copy(zero_v, acc.at[pl.ds(s * 6256 + p * ZROWS, ZROWS)])
        plsc.subcore_barrier()

        base = wid * BPS

        def fire_idx(g, par):
            gb = base + g * GRP
            pltpu.async_copy(srcb.at[pl.ds(gb, GRP)], src_v.at[par],
                             isem[par])
            pltpu.async_copy(dstb.at[pl.ds(gb, GRP)], dst_v.at[par],
                             isem[par])

        def wait_idx(par):
            pltpu.make_async_copy(srcb.at[pl.ds(0, GRP)], src_v.at[par],
                                  isem[par]).wait()
            pltpu.make_async_copy(dstb.at[pl.ds(0, GRP)], dst_v.at[par],
                                  isem[par]).wait()

        def wait_rows(sem, par):
            pltpu.make_async_copy(table.at[pl.ds(0, GRP * 128)],
                                  rows_v.at[par], sem).wait()

        def stage(par, g):
            opp = 1 - par
            wait_idx(par)
            for b in range(GRP):
                pltpu.async_copy(table.at[src_v.at[par, b]],
                                 rows_v.at[par, pl.ds(b * 128, 128)],
                                 gsem[par])

            @pl.when(g >= 1)
            def _drain_prev():
                wait_rows(ssem[opp], opp)

            @pl.when(g + 1 < GROUPS)
            def _prefetch():
                fire_idx(g + 1, opp)

            wait_rows(gsem[par], par)
            for b in range(GRP):
                pltpu.async_copy(rows_v.at[par, pl.ds(b * 128, 128)],
                                 acc.at[dst_v.at[par, b]],
                                 ssem[par], add=True)

        fire_idx(0, 0)

        def body(i, carry):
            stage(0, 2 * i)

            @pl.when(2 * i + 1 < GROUPS)
            def _odd():
                stage(1, 2 * i + 1)

            return carry

        lax.fori_loop(0, (GROUPS + 1) // 2, body, 0)
        wait_rows(ssem[(GROUPS - 1) % 2], (GROUPS - 1) % 2)
        plsc.subcore_barrier()
        @pl.when(s < 15)
        def _wb_main():
            pltpu.sync_copy(acc.at[pl.ds(s * 6256, 6256)],
                            out.at[pl.ds(c * N + s * 6256, 6256)])

        @pl.when(s == 15)
        def _wb_tail():
            pltpu.sync_copy(acc.at[pl.ds(15 * 6256, 6160)],
                            out.at[pl.ds(c * N + 15 * 6256, 6160)])

    return agg


_AGG_CACHE = []


def _agg_call(table, srcb, dstb):
    if not _AGG_CACHE:
        _AGG_CACHE.append(_make_agg())
    return _AGG_CACHE[0](table, srcb, dstb)


def _plo_spec():
    return pl.BlockSpec((BN, CW), lambda i: (i, 0))


def _phi_spec():
    return pl.BlockSpec((BN, CW), lambda i: (i + NB, 0))


def _row_spec():
    return pl.BlockSpec((BN, CW), lambda i: (i, 0))


def _full_spec(shape):
    return pl.BlockSpec(shape, lambda i: tuple(0 for _ in shape))


def _mlp0_body(plo_ref, phi_ref, x_ref, w1_ref, b1_ref, w2_ref, b2_ref,
               *o_refs):
    cat = plo_ref[...] + phi_ref[...] + x_ref[...]
    t = _hi(cat, w1_ref[...]) + b1_ref[...]
    t = _leaky(t)
    t = _hi(t, w2_ref[...]) + b2_ref[...]
    t = _leaky(t)
    for cc in range(NCH):
        o_refs[cc][...] = t[:, cc * CW:(cc + 1) * CW]


def _mlp_body(trailing, *refs):
    ps = refs[0:2 * NCH]
    hs = refs[2 * NCH:3 * NCH]
    w1_ref, b1_ref, w2_ref, b2_ref = refs[3 * NCH:3 * NCH + 4]
    o_refs = refs[3 * NCH + 4:]
    cat = jnp.concatenate(
        [ps[2 * cc][...] + ps[2 * cc + 1][...] + hs[cc][...]
         for cc in range(NCH)], axis=1)
    t = _hi(cat, w1_ref[...]) + b1_ref[...]
    t = _leaky(t)
    t = _hi(t, w2_ref[...]) + b2_ref[...]
    if trailing:
        t = _leaky(t)
    for cc in range(NCH):
        o_refs[cc][...] = t[:, cc * CW:(cc + 1) * CW]


def _final_body(*refs):
    ps = refs[0:2 * NCH]
    hs = refs[2 * NCH:3 * NCH]
    w1_ref, b1_ref, w2_ref, b2_ref, batch_ref, wl_ref, bl_ref = \
        refs[3 * NCH:3 * NCH + 7]
    o_ref = refs[3 * NCH + 7]
    sums, counts = refs[3 * NCH + 8:]
    i = pl.program_id(0)

    @pl.when(i == 0)
    def _init():
        sums[...] = jnp.zeros_like(sums)
        counts[...] = jnp.zeros_like(counts)

    cat = jnp.concatenate(
        [ps[2 * cc][...] + ps[2 * cc + 1][...] + hs[cc][...]
         for cc in range(NCH)], axis=1)
    t = _hi(cat, w1_ref[...]) + b1_ref[...]
    t = _leaky(t)
    t = _hi(t, w2_ref[...]) + b2_ref[...]
    z = jnp.maximum(t, 0.0)
    b = batch_ref[0, 0, :]
    onehot = (b[:, None] ==
              lax.broadcasted_iota(jnp.int32, (BN, G), 1)).astype(jnp.float32)
    sums[...] += lax.dot_general(onehot, z, (((0,), (0,)), ((), ())),
                                 precision=lax.Precision.HIGHEST)
    counts[...] += lax.dot_general(onehot, jnp.ones((BN, 1), jnp.float32),
                                   (((0,), (0,)), ((), ())),
                                   precision=lax.Precision.HIGHEST)

    @pl.when(i == NB - 1)
    def _fin():
        mean = sums[...] / jnp.maximum(counts[...], 1.0)
        logits = _hi(mean, wl_ref[...]) + bl_ref[...]
        m = jnp.max(logits, axis=1, keepdims=True)
        lse = m + jnp.log(jnp.sum(jnp.exp(logits - m), axis=1, keepdims=True))
        o_ref[...] = logits - lse


def _mlp0_call(p0, x16, w1e, b1, w2, b2):
    return pl.pallas_call(
        _mlp0_body,
        grid=(NB,),
        in_specs=[
            _plo_spec(), _phi_spec(), _row_spec(),
            _full_spec((CW, H)), _full_spec((1, H)),
            _full_spec((H, H)), _full_spec((1, H)),
        ],
        out_specs=[_row_spec() for _ in range(NCH)],
        out_shape=[jax.ShapeDtypeStruct((N, CW), jnp.float32)
                   for _ in range(NCH)],
    )(p0, p0, x16, w1e, b1, w2, b2)


def _mlp_call(ps, hs, w1, b1, w2, b2, trailing):
    return pl.pallas_call(
        functools.partial(_mlp_body, trailing),
        grid=(NB,),
        in_specs=(
            [sp for _ in range(NCH) for sp in (_plo_spec(), _phi_spec())]
            + [_row_spec() for _ in range(NCH)]
            + [_full_spec((H, H)), _full_spec((1, H)),
               _full_spec((H, H)), _full_spec((1, H))]
        ),
        out_specs=[_row_spec() for _ in range(NCH)],
        out_shape=[jax.ShapeDtypeStruct((N, CW), jnp.float32)
                   for _ in range(NCH)],
    )(*[p for pp in ps for p in (pp, pp)], *hs, w1, b1, w2, b2)


def _final_call(ps, hs, w1, b1, w2, b2, batch3d, wl, bl):
    return pl.pallas_call(
        _final_body,
        grid=(NB,),
        in_specs=(
            [sp for _ in range(NCH) for sp in (_plo_spec(), _phi_spec())]
            + [_row_spec() for _ in range(NCH)]
            + [_full_spec((H, H)), _full_spec((1, H)),
               _full_spec((H, H)), _full_spec((1, H)),
               pl.BlockSpec((1, 1, BN), lambda i: (i, 0, 0)),
               _full_spec((H, 2)), _full_spec((1, 2))]
        ),
        out_specs=pl.BlockSpec((G, 2), lambda i: (0, 0)),
        out_shape=jax.ShapeDtypeStruct((G, 2), jnp.float32),
        scratch_shapes=[pltpu.VMEM((G, H), jnp.float32),
                        pltpu.VMEM((G, 1), jnp.float32)],
        compiler_params=pltpu.CompilerParams(
            dimension_semantics=("arbitrary",)),
    )(*[p for pp in ps for p in (pp, pp)], *hs, w1, b1, w2, b2,
      batch3d, wl, bl)


def kernel(x, edge_index, batch, W1_0, b1_0, W2_0, b2_0, W1_1, b1_1, W2_1,
           b2_1, W1_2, b1_2, W2_2, b2_2, W1_3, b1_3, W2_3, b2_3, Wl, bl):
    src = edge_index[0]
    dst = edge_index[1]
    pad = EPAD - E
    srcb = jnp.concatenate(
        [src, jnp.zeros((pad,), jnp.int32)]).reshape(BATCHES, 128)
    dstb = jnp.concatenate(
        [dst, jnp.full((pad,), N, jnp.int32)]).reshape(BATCHES, 128)
    x16 = jnp.tile(x, (1, CW))
    batch3d = batch.reshape(NB, 1, BN)
    w1e = jnp.concatenate([W1_0, jnp.zeros((CW - 1, H), jnp.float32)], axis=0)
    rb = lambda v: v.reshape(1, -1)

    p0 = _agg_call(x16, srcb, dstb)
    hs = _mlp0_call(p0, x16, w1e, rb(b1_0), W2_0, rb(b2_0))
    for (w1, b1, w2, b2) in ((W1_1, b1_1, W2_1, b2_1),
                             (W1_2, b1_2, W2_2, b2_2)):
        ps = [_agg_call(hs[cc], srcb, dstb) for cc in range(NCH)]
        hs = _mlp_call(ps, hs, w1, rb(b1), w2, rb(b2), trailing=True)
    ps = [_agg_call(hs[cc], srcb, dstb) for cc in range(NCH)]
    return _final_call(ps, hs, W1_3, rb(b1_3), W2_3, rb(b2_3),
                       batch3d, Wl, rb(bl))
